# baseline (speedup 1.0000x reference)
import functools

import jax
import jax.numpy as jnp
from jax.experimental import pallas as pl

N = 10000
E = 320000
H = 256
BE = 2560


def _edge_mm_kernel(e_ref, wm_ref, we_ref, em_ref, ee_ref):
    e = e_ref[...]
    em_ref[...] = jnp.dot(e, wm_ref[...], preferred_element_type=jnp.float32)
    ee_ref[...] = jnp.dot(e, we_ref[...], preferred_element_type=jnp.float32)


@functools.partial(jax.jit, static_argnames=())
def _edge_mm(e, w_msg, w_ee):
    grid = (E // BE,)
    return pl.pallas_call(
        _edge_mm_kernel,
        grid=grid,
        in_specs=[
            pl.BlockSpec((BE, H), lambda i: (i, 0)),
            pl.BlockSpec((H, H), lambda i: (0, 0)),
            pl.BlockSpec((H, H), lambda i: (0, 0)),
        ],
        out_specs=[
            pl.BlockSpec((BE, H), lambda i: (i, 0)),
            pl.BlockSpec((BE, H), lambda i: (i, 0)),
        ],
        out_shape=[
            jax.ShapeDtypeStruct((E, H), jnp.float32),
            jax.ShapeDtypeStruct((E, H), jnp.float32),
        ],
    )(e, w_msg, w_ee)


def kernel(x, edge_index, edge_attr, params):
    src = edge_index[0]
    dst = edge_index[1]
    h = x @ params['W_ne'] + params['b_ne']
    e = edge_attr @ params['W_ee'] + params['b_ee']
    for l in range(3):
        p = params['layers'][l]
        we_s, we_d, we_e = jnp.split(p['W_e'], 3, axis=0)
        em, ee = _edge_mm(e, p['W_msg'], we_e)
        hm = h @ p['W_msg']
        m = jnp.take(hm, src, axis=0) + em
        agg = jax.ops.segment_sum(m, dst, num_segments=N)
        h = jax.nn.relu(h @ p['W_self'] + agg + p['b_h'])
        hs = h @ we_s
        hd = h @ we_d
        e = jax.nn.relu(
            jnp.take(hs, src, axis=0) + jnp.take(hd, dst, axis=0) + ee + p['b_e'])
    return h @ params['W_pred'] + params['b_pred']


# speedup vs baseline: 1.4802x; 1.4802x over previous
import functools

import jax
import jax.numpy as jnp
from jax import lax
from jax.experimental import pallas as pl
from jax.experimental.pallas import tpu as pltpu
from jax.experimental.pallas import tpu_sc as plsc

N = 10000
E = 320000
H = 256
BE = 2560

NS = 16
EPT = E // NS
K = 80
NPAD = 10240
RPT = NPAD // NS


def _edge_mm_kernel(e_ref, wm_ref, we_ref, em_ref, ee_ref):
    e = e_ref[...]
    em_ref[...] = jnp.dot(e, wm_ref[...], preferred_element_type=jnp.float32)
    ee_ref[...] = jnp.dot(e, we_ref[...], preferred_element_type=jnp.float32)


def _edge_mm(e, w_msg, w_ee):
    return pl.pallas_call(
        _edge_mm_kernel,
        grid=(E // BE,),
        in_specs=[
            pl.BlockSpec((BE, H), lambda i: (i, 0)),
            pl.BlockSpec((H, H), lambda i: (0, 0)),
            pl.BlockSpec((H, H), lambda i: (0, 0)),
        ],
        out_specs=[
            pl.BlockSpec((BE, H), lambda i: (i, 0)),
            pl.BlockSpec((BE, H), lambda i: (i, 0)),
        ],
        out_shape=[
            jax.ShapeDtypeStruct((E, H), jnp.float32),
            jax.ShapeDtypeStruct((E, H), jnp.float32),
        ],
    )(e, w_msg, w_ee)


_sc_mesh = plsc.VectorSubcoreMesh(core_axis_name="c", subcore_axis_name="s")


@functools.partial(
    pl.kernel,
    out_type=jax.ShapeDtypeStruct((2 * NPAD, 128), jnp.float32),
    mesh=_sc_mesh,
    scratch_types=[
        pltpu.VMEM((K,), jnp.int32),
        pltpu.VMEM((K,), jnp.int32),
        pltpu.VMEM((K, 128), jnp.float32),
        pltpu.VMEM((K, 128), jnp.float32),
        pltpu.VMEM_SHARED((NPAD, 128), jnp.float32),
        pltpu.SemaphoreType.DMA,
    ],
)
def _sc_scatter(hm_hbm, em_hbm, src2_hbm, dst_hbm, z_hbm, out_hbm,
                sidx, didx, grows, erows, acc, sem):
    c = lax.axis_index("c")
    s = lax.axis_index("s")
    pltpu.sync_copy(z_hbm, acc.at[pl.ds(s * RPT, RPT)])
    plsc.subcore_barrier()

    @pl.loop(0, EPT, step=K)
    def _(i):
        b = s * EPT + i
        pltpu.sync_copy(src2_hbm.at[pl.ds(c * E + b, K)], sidx)
        pltpu.sync_copy(dst_hbm.at[pl.ds(b, K)], didx)
        pltpu.async_copy(hm_hbm.at[sidx], grows, sem).wait()
        pltpu.sync_copy(em_hbm.at[pl.ds(b, K), pl.ds(c * 128, 128)], erows)
        pltpu.sync_copy(grows, acc.at[didx], add=True)
        pltpu.sync_copy(erows, acc.at[didx], add=True)

    plsc.subcore_barrier()
    pltpu.sync_copy(acc.at[pl.ds(s * RPT, RPT)],
                    out_hbm.at[pl.ds(c * NPAD + s * RPT, RPT)])


def kernel(x, edge_index, edge_attr, params):
    src = edge_index[0]
    dst = edge_index[1]
    src2 = jnp.concatenate([src, src + N])
    zrows = jnp.zeros((RPT, 128), jnp.float32)
    h = x @ params['W_ne'] + params['b_ne']
    e = edge_attr @ params['W_ee'] + params['b_ee']
    for l in range(3):
        p = params['layers'][l]
        we_s, we_d, we_e = jnp.split(p['W_e'], 3, axis=0)
        em, ee = _edge_mm(e, p['W_msg'], we_e)
        hm = h @ p['W_msg']
        hm_flat = jnp.concatenate([hm[:, :128], hm[:, 128:]], axis=0)
        agg_flat = _sc_scatter(hm_flat, em, src2, dst, zrows)
        agg = jnp.concatenate(
            [agg_flat[0:N], agg_flat[NPAD:NPAD + N]], axis=1)
        h = jax.nn.relu(h @ p['W_self'] + agg + p['b_h'])
        hs = h @ we_s
        hd = h @ we_d
        e = jax.nn.relu(
            jnp.take(hs, src, axis=0) + jnp.take(hd, dst, axis=0) + ee + p['b_e'])
    return h @ params['W_pred'] + params['b_pred']


# speedup vs baseline: 1.8534x; 1.2522x over previous
import functools

import jax
import jax.numpy as jnp
from jax import lax
from jax.experimental import pallas as pl
from jax.experimental.pallas import tpu as pltpu
from jax.experimental.pallas import tpu_sc as plsc

N = 10000
E = 320000
H = 256
BE = 2560

NS = 16
EPT = E // NS
K = 80
NPAD = 10240
RPT = NPAD // NS


def _edge_mm0_kernel(ea_ref, wm_ref, we_ref, bm_ref, be_ref, em_ref, ee_ref):
    ea = ea_ref[...]
    em_ref[...] = jnp.dot(ea, wm_ref[...],
                          preferred_element_type=jnp.float32) + bm_ref[...]
    ee_ref[...] = jnp.dot(ea, we_ref[...],
                          preferred_element_type=jnp.float32) + be_ref[...]


def _edge_mm0(ea, wcm, wce, bcm, bce):
    de = ea.shape[1]
    return pl.pallas_call(
        _edge_mm0_kernel,
        grid=(E // BE,),
        in_specs=[
            pl.BlockSpec((BE, de), lambda i: (i, 0)),
            pl.BlockSpec((de, H), lambda i: (0, 0)),
            pl.BlockSpec((de, H), lambda i: (0, 0)),
            pl.BlockSpec((1, H), lambda i: (0, 0)),
            pl.BlockSpec((1, H), lambda i: (0, 0)),
        ],
        out_specs=[
            pl.BlockSpec((BE, H), lambda i: (i, 0)),
            pl.BlockSpec((BE, H), lambda i: (i, 0)),
        ],
        out_shape=[
            jax.ShapeDtypeStruct((E, H), jnp.float32),
            jax.ShapeDtypeStruct((E, H), jnp.float32),
        ],
    )(ea, wcm, wce, bcm, bce)


def _edge_mm_kernel(ee_ref, ga_ref, gb_ref, be_ref, wm_ref, we_ref,
                    em_ref, eeo_ref):
    e = jax.nn.relu(
        ee_ref[...]
        + jnp.concatenate([ga_ref[...], gb_ref[...]], axis=1)
        + be_ref[...])
    em_ref[...] = jnp.dot(e, wm_ref[...], preferred_element_type=jnp.float32)
    if eeo_ref is not None:
        eeo_ref[...] = jnp.dot(e, we_ref[...],
                               preferred_element_type=jnp.float32)


def _edge_mm(ee_prev, g_flat, b_e, w_msg, w_ee, want_ee):
    nblk = E // BE
    out_specs = [pl.BlockSpec((BE, H), lambda i: (i, 0))]
    out_shape = [jax.ShapeDtypeStruct((E, H), jnp.float32)]
    if want_ee:
        out_specs.append(pl.BlockSpec((BE, H), lambda i: (i, 0)))
        out_shape.append(jax.ShapeDtypeStruct((E, H), jnp.float32))
        body = _edge_mm_kernel
    else:
        body = functools.partial(_edge_mm_kernel, eeo_ref=None)
    res = pl.pallas_call(
        body,
        grid=(nblk,),
        in_specs=[
            pl.BlockSpec((BE, H), lambda i: (i, 0)),
            pl.BlockSpec((BE, 128), lambda i: (i, 0)),
            pl.BlockSpec((BE, 128), lambda i: (i + nblk, 0)),
            pl.BlockSpec((1, H), lambda i: (0, 0)),
            pl.BlockSpec((H, H), lambda i: (0, 0)),
            pl.BlockSpec((H, H), lambda i: (0, 0)),
        ],
        out_specs=out_specs,
        out_shape=out_shape,
    )(ee_prev, g_flat, g_flat, b_e, w_msg, w_ee)
    return res if want_ee else (res[0], None)


_sc_mesh = plsc.VectorSubcoreMesh(core_axis_name="c", subcore_axis_name="s")


@functools.partial(
    pl.kernel,
    out_type=jax.ShapeDtypeStruct((2 * NPAD, 128), jnp.float32),
    mesh=_sc_mesh,
    scratch_types=[
        pltpu.VMEM((K,), jnp.int32),
        pltpu.VMEM((K,), jnp.int32),
        pltpu.VMEM((K, 128), jnp.float32),
        pltpu.VMEM((K, 128), jnp.float32),
        pltpu.VMEM_SHARED((NPAD, 128), jnp.float32),
        pltpu.SemaphoreType.DMA,
    ],
)
def _sc_scatter(hm_hbm, em_hbm, src2_hbm, dst_hbm, z_hbm, out_hbm,
                sidx, didx, grows, erows, acc, sem):
    c = lax.axis_index("c")
    s = lax.axis_index("s")
    pltpu.sync_copy(z_hbm, acc.at[pl.ds(s * RPT, RPT)])
    plsc.subcore_barrier()

    @pl.loop(0, EPT, step=K)
    def _(i):
        b = s * EPT + i
        pltpu.sync_copy(src2_hbm.at[pl.ds(c * E + b, K)], sidx)
        pltpu.sync_copy(dst_hbm.at[pl.ds(b, K)], didx)
        pltpu.async_copy(hm_hbm.at[sidx], grows, sem).wait()
        pltpu.sync_copy(em_hbm.at[pl.ds(b, K), pl.ds(c * 128, 128)], erows)
        pltpu.sync_copy(grows, acc.at[didx], add=True)
        pltpu.sync_copy(erows, acc.at[didx], add=True)

    plsc.subcore_barrier()
    pltpu.sync_copy(acc.at[pl.ds(s * RPT, RPT)],
                    out_hbm.at[pl.ds(c * NPAD + s * RPT, RPT)])


@functools.partial(
    pl.kernel,
    out_type=jax.ShapeDtypeStruct((2 * E, 128), jnp.float32),
    mesh=_sc_mesh,
    scratch_types=[
        pltpu.VMEM((K,), jnp.int32),
        pltpu.VMEM((K,), jnp.int32),
        pltpu.VMEM((K,), jnp.int32),
        pltpu.VMEM((K, 128), jnp.float32),
        pltpu.VMEM((K, 128), jnp.float32),
        pltpu.VMEM_SHARED((NS * K, 128), jnp.float32),
        pltpu.SemaphoreType.DMA,
    ],
)
def _sc_gather(hs_hbm, hd_hbm, src2_hbm, dst2_hbm, ident_hbm, out_hbm,
               sidx, didx, ident, rows_a, rows_b, spm, sem):
    c = lax.axis_index("c")
    s = lax.axis_index("s")
    pltpu.sync_copy(ident_hbm.at[pl.ds(s * K, K)], ident)

    @pl.loop(0, EPT, step=K)
    def _(i):
        b = s * EPT + i
        pltpu.sync_copy(src2_hbm.at[pl.ds(c * E + b, K)], sidx)
        pltpu.sync_copy(dst2_hbm.at[pl.ds(c * E + b, K)], didx)
        pltpu.async_copy(hs_hbm.at[sidx], rows_a, sem).wait()
        pltpu.async_copy(hd_hbm.at[didx], rows_b, sem).wait()
        pltpu.sync_copy(rows_a, spm.at[pl.ds(s * K, K)])
        pltpu.sync_copy(rows_b, spm.at[ident], add=True)
        pltpu.sync_copy(spm.at[pl.ds(s * K, K)], out_hbm.at[pl.ds(c * E + b, K)])


def _split_cols(t):
    return jnp.concatenate([t[:, :128], t[:, 128:]], axis=0)


def kernel(x, edge_index, edge_attr, params):
    src = edge_index[0]
    dst = edge_index[1]
    src2 = jnp.concatenate([src, src + N])
    dst2 = jnp.concatenate([dst, dst + N])
    zrows = jnp.zeros((RPT, 128), jnp.float32)
    ident = jnp.arange(NS * K, dtype=jnp.int32)
    h = x @ params['W_ne'] + params['b_ne']
    em = ee = None
    for l in range(3):
        p = params['layers'][l]
        we_s, we_d, we_e = jnp.split(p['W_e'], 3, axis=0)
        if l == 0:
            wcm = params['W_ee'] @ p['W_msg']
            wce = params['W_ee'] @ we_e
            bcm = (params['b_ee'] @ p['W_msg']).reshape(1, H)
            bce = (params['b_ee'] @ we_e).reshape(1, H)
            em, ee = _edge_mm0(edge_attr, wcm, wce, bcm, bce)
        else:
            em, ee = _edge_mm(ee, g_flat, params['layers'][l - 1]['b_e']
                              .reshape(1, H), p['W_msg'], we_e,
                              want_ee=(l < 2))
        hm = h @ p['W_msg']
        agg_flat = _sc_scatter(_split_cols(hm), em, src2, dst, zrows)
        agg = jnp.concatenate(
            [agg_flat[0:N], agg_flat[NPAD:NPAD + N]], axis=1)
        h = jax.nn.relu(h @ p['W_self'] + agg + p['b_h'])
        if l < 2:
            hs = h @ we_s
            hd = h @ we_d
            g_flat = _sc_gather(_split_cols(hs), _split_cols(hd),
                                src2, dst2, ident)
    return h @ params['W_pred'] + params['b_pred']


# speedup vs baseline: 3.9408x; 2.1263x over previous
import functools

import jax
import jax.numpy as jnp
from jax import lax
from jax.experimental import pallas as pl
from jax.experimental.pallas import tpu as pltpu
from jax.experimental.pallas import tpu_sc as plsc

N = 10000
E = 320000
H = 256
BE = 2560

NS = 16
NPAD = 10240
RPT = NPAD // NS

KS = 80
SCH = (E // NS) // KS

_sc_mesh = plsc.VectorSubcoreMesh(core_axis_name="c", subcore_axis_name="s")


def _edge_mm0_kernel(ea_ref, wm_ref, we_ref, bm_ref, be_ref, em_ref, ee_ref):
    ea = ea_ref[...]
    em_ref[...] = jnp.dot(ea, wm_ref[...],
                          preferred_element_type=jnp.float32) + bm_ref[...]
    ee_ref[...] = jnp.dot(ea, we_ref[...],
                          preferred_element_type=jnp.float32) + be_ref[...]


def _edge_mm0(ea, wcm, wce, bcm, bce):
    de = ea.shape[1]
    return pl.pallas_call(
        _edge_mm0_kernel,
        grid=(E // BE,),
        in_specs=[
            pl.BlockSpec((BE, de), lambda i: (i, 0)),
            pl.BlockSpec((de, H), lambda i: (0, 0)),
            pl.BlockSpec((de, H), lambda i: (0, 0)),
            pl.BlockSpec((1, H), lambda i: (0, 0)),
            pl.BlockSpec((1, H), lambda i: (0, 0)),
        ],
        out_specs=[
            pl.BlockSpec((BE, H), lambda i: (i, 0)),
            pl.BlockSpec((BE, H), lambda i: (i, 0)),
        ],
        out_shape=[
            jax.ShapeDtypeStruct((E, H), jnp.float32),
            jax.ShapeDtypeStruct((E, H), jnp.float32),
        ],
    )(ea, wcm, wce, bcm, bce)


def _edge_mm_kernel(ee_ref, ga_ref, gb_ref, be_ref, wm_ref, we_ref,
                    em_ref, eeo_ref):
    e = jax.nn.relu(
        ee_ref[...]
        + jnp.concatenate([ga_ref[...], gb_ref[...]], axis=1)
        + be_ref[...])
    em_ref[...] = jnp.dot(e, wm_ref[...], preferred_element_type=jnp.float32)
    if eeo_ref is not None:
        eeo_ref[...] = jnp.dot(e, we_ref[...],
                               preferred_element_type=jnp.float32)


def _edge_mm(ee_prev, g_flat, b_e, w_msg, w_ee, want_ee):
    nblk = E // BE
    out_specs = [pl.BlockSpec((BE, H), lambda i: (i, 0))]
    out_shape = [jax.ShapeDtypeStruct((E, H), jnp.float32)]
    if want_ee:
        out_specs.append(pl.BlockSpec((BE, H), lambda i: (i, 0)))
        out_shape.append(jax.ShapeDtypeStruct((E, H), jnp.float32))
        body = _edge_mm_kernel
    else:
        body = functools.partial(_edge_mm_kernel, eeo_ref=None)
    res = pl.pallas_call(
        body,
        grid=(nblk,),
        in_specs=[
            pl.BlockSpec((BE, H), lambda i: (i, 0)),
            pl.BlockSpec((BE, 128), lambda i: (i, 0)),
            pl.BlockSpec((BE, 128), lambda i: (i + nblk, 0)),
            pl.BlockSpec((1, H), lambda i: (0, 0)),
            pl.BlockSpec((H, H), lambda i: (0, 0)),
            pl.BlockSpec((H, H), lambda i: (0, 0)),
        ],
        out_specs=out_specs,
        out_shape=out_shape,
    )(ee_prev, g_flat, g_flat, b_e, w_msg, w_ee)
    return res if want_ee else (res[0], None)


def _off(v, m):
    return pl.multiple_of(v, m)


@functools.partial(
    pl.kernel,
    out_type=jax.ShapeDtypeStruct((2 * NPAD, 128), jnp.float32),
    mesh=_sc_mesh,
    scratch_types=[
        pltpu.VMEM((KS,), jnp.int32),
        pltpu.VMEM((KS,), jnp.int32),
        pltpu.VMEM((KS,), jnp.int32),
        pltpu.VMEM((KS,), jnp.int32),
        pltpu.VMEM((KS, 128), jnp.float32),
        pltpu.VMEM((KS, 128), jnp.float32),
        pltpu.VMEM((KS, 128), jnp.float32),
        pltpu.VMEM((KS, 128), jnp.float32),
        pltpu.VMEM_SHARED((NPAD, 128), jnp.float32),
    ] + [pltpu.SemaphoreType.DMA] * 8,
)
def _sc_scatter(hm_hbm, em_hbm, src2_hbm, dst_hbm, z_hbm, out_hbm,
                si0, si1, di0, di1, ga0, ga1, em0, em1, acc,
                ssi0, ssi1, sdi0, sdi1, sg0, sg1, se0, se1):
    c = lax.axis_index("c")
    s = lax.axis_index("s")
    pltpu.sync_copy(z_hbm, acc.at[pl.ds(s * RPT, RPT)])

    tbase = (c * NS + s) * SCH
    dbase = s * SCH

    def sioff(j):
        return _off(jnp.minimum(tbase + j, tbase + SCH - 1) * KS, 8)

    def dioff(j):
        return _off(jnp.minimum(dbase + j, dbase + SCH - 1) * KS, 8)

    def si_start(j, si, sem):
        pltpu.async_copy(src2_hbm.at[pl.ds(sioff(j), KS)], si, sem)

    def si_wait(j, si, sem):
        pltpu.make_async_copy(src2_hbm.at[pl.ds(sioff(j), KS)], si, sem).wait()

    def di_start(j, di, sem):
        pltpu.async_copy(dst_hbm.at[pl.ds(dioff(j), KS)], di, sem)

    def di_wait(j, di, sem):
        pltpu.make_async_copy(dst_hbm.at[pl.ds(dioff(j), KS)], di, sem).wait()

    def g_start(j, si, ga, em, sg, se):
        pltpu.async_copy(hm_hbm.at[si], ga, sg)
        pltpu.async_copy(
            em_hbm.at[pl.ds(dioff(j), KS), pl.ds(_off(c * 128, 128), 128)],
            em, se)

    def g_wait(j, si, ga, em, sg, se):
        pltpu.make_async_copy(hm_hbm.at[si], ga, sg).wait()
        pltpu.make_async_copy(
            em_hbm.at[pl.ds(dioff(j), KS), pl.ds(_off(c * 128, 128), 128)],
            em, se).wait()

    si_start(0, si0, ssi0)
    si_start(1, si1, ssi1)
    di_start(0, di0, sdi0)
    di_start(1, di1, sdi1)
    si_wait(0, si0, ssi0)
    g_start(0, si0, ga0, em0, sg0, se0)
    si_wait(1, si1, ssi1)
    g_start(1, si1, ga1, em1, sg1, se1)
    plsc.subcore_barrier()

    def chunk(j, si, di, ga, em, ssi, sdi, sg, se, jn):
        g_wait(j, si, ga, em, sg, se)
        si_start(jn, si, ssi)
        di_wait(j, di, sdi)
        pltpu.sync_copy(ga, acc.at[di], add=True)
        pltpu.sync_copy(em, acc.at[di], add=True)
        di_start(jn, di, sdi)
        si_wait(jn, si, ssi)
        g_start(jn, si, ga, em, sg, se)

    @pl.loop(0, SCH, step=2)
    def _(j):
        chunk(j, si0, di0, ga0, em0, ssi0, sdi0, sg0, se0, j + 2)
        chunk(j + 1, si1, di1, ga1, em1, ssi1, sdi1, sg1, se1, j + 3)

    g_wait(SCH, si0, ga0, em0, sg0, se0)
    g_wait(SCH + 1, si1, ga1, em1, sg1, se1)
    di_wait(SCH, di0, sdi0)
    di_wait(SCH + 1, di1, sdi1)

    plsc.subcore_barrier()
    pltpu.sync_copy(acc.at[pl.ds(s * RPT, RPT)],
                    out_hbm.at[pl.ds(c * NPAD + s * RPT, RPT)])


@functools.partial(
    pl.kernel,
    out_type=jax.ShapeDtypeStruct((2 * E, 128), jnp.float32),
    mesh=_sc_mesh,
    scratch_types=[
        pltpu.VMEM((KS,), jnp.int32),
        pltpu.VMEM((KS,), jnp.int32),
        pltpu.VMEM((KS,), jnp.int32),
        pltpu.VMEM((KS,), jnp.int32),
        pltpu.VMEM((KS,), jnp.int32),
        pltpu.VMEM((KS,), jnp.int32),
        pltpu.VMEM((KS, 128), jnp.float32),
        pltpu.VMEM((KS, 128), jnp.float32),
        pltpu.VMEM((KS, 128), jnp.float32),
        pltpu.VMEM((KS, 128), jnp.float32),
        pltpu.VMEM_SHARED((2 * NS * KS, 128), jnp.float32),
    ] + [pltpu.SemaphoreType.DMA] * 10,
)
def _sc_gather(hs_hbm, hd_hbm, src2_hbm, dst2_hbm, ident_hbm, out_hbm,
               si0, si1, di0, di1, id0, id1, a0, a1, b0, b1, spm,
               ssi0, ssi1, sdi0, sdi1, sa0, sa1, sb0, sb1, sf0, sf1):
    c = lax.axis_index("c")
    s = lax.axis_index("s")
    tbase = (c * NS + s) * SCH
    pltpu.sync_copy(ident_hbm.at[pl.ds(_off(s * 2 * KS, 8), KS)], id0)
    pltpu.sync_copy(ident_hbm.at[pl.ds(_off(s * 2 * KS + KS, 8), KS)], id1)

    def ioff(j):
        return _off(jnp.minimum(tbase + j, tbase + SCH - 1) * KS, 8)

    def i_start(j, si, di, ssi, sdi):
        pltpu.async_copy(src2_hbm.at[pl.ds(ioff(j), KS)], si, ssi)
        pltpu.async_copy(dst2_hbm.at[pl.ds(ioff(j), KS)], di, sdi)

    def i_wait(j, si, di, ssi, sdi):
        pltpu.make_async_copy(src2_hbm.at[pl.ds(ioff(j), KS)], si, ssi).wait()
        pltpu.make_async_copy(dst2_hbm.at[pl.ds(ioff(j), KS)], di, sdi).wait()

    def g_start(si, di, a, b, sa, sb):
        pltpu.async_copy(hs_hbm.at[si], a, sa)
        pltpu.async_copy(hd_hbm.at[di], b, sb)

    def g_wait(si, di, a, b, sa, sb):
        pltpu.make_async_copy(hs_hbm.at[si], a, sa).wait()
        pltpu.make_async_copy(hd_hbm.at[di], b, sb).wait()

    def slot(p):
        return spm.at[pl.ds(_off((s * 2 + p) * KS, 8), KS)]

    def f_start(p, j, sf):
        pltpu.async_copy(slot(p), out_hbm.at[pl.ds(ioff(j), KS)], sf)

    def f_wait(p, j, sf):
        pltpu.make_async_copy(
            slot(p), out_hbm.at[pl.ds(ioff(j), KS)], sf).wait()

    def process(p, j, a, b, idp, sf):
        pltpu.sync_copy(a, slot(p))
        pltpu.sync_copy(b, spm.at[idp], add=True)
        f_start(p, j, sf)

    def chunk(p, j, si, di, idp, a, b, ssi, sdi, sa, sb, sf, wait_f):
        g_wait(si, di, a, b, sa, sb)
        i_start(j + 2, si, di, ssi, sdi)
        if wait_f:
            f_wait(p, j, sf)
        process(p, j, a, b, idp, sf)
        i_wait(j + 2, si, di, ssi, sdi)
        g_start(si, di, a, b, sa, sb)

    i_start(0, si0, di0, ssi0, sdi0)
    i_start(1, si1, di1, ssi1, sdi1)
    i_wait(0, si0, di0, ssi0, sdi0)
    g_start(si0, di0, a0, b0, sa0, sb0)
    i_wait(1, si1, di1, ssi1, sdi1)
    g_start(si1, di1, a1, b1, sa1, sb1)

    chunk(0, 0, si0, di0, id0, a0, b0, ssi0, sdi0, sa0, sb0, sf0, False)
    chunk(1, 1, si1, di1, id1, a1, b1, ssi1, sdi1, sa1, sb1, sf1, False)

    @pl.loop(2, SCH, step=2)
    def _(j):
        chunk(0, j, si0, di0, id0, a0, b0, ssi0, sdi0, sa0, sb0, sf0, True)
        chunk(1, j + 1, si1, di1, id1, a1, b1, ssi1, sdi1, sa1, sb1, sf1, True)

    g_wait(si0, di0, a0, b0, sa0, sb0)
    g_wait(si1, di1, a1, b1, sa1, sb1)
    f_wait(0, SCH - 2, sf0)
    f_wait(1, SCH - 1, sf1)


def _split_cols(t):
    return jnp.concatenate([t[:, :128], t[:, 128:]], axis=0)


def kernel(x, edge_index, edge_attr, params):
    src = edge_index[0]
    dst = edge_index[1]
    src2 = jnp.concatenate([src, src + N])
    dst2 = jnp.concatenate([dst, dst + N])
    zrows = jnp.zeros((RPT, 128), jnp.float32)
    ident = jnp.arange(2 * NS * KS, dtype=jnp.int32)
    h = x @ params['W_ne'] + params['b_ne']
    em = ee = g = None
    for l in range(3):
        p = params['layers'][l]
        we_s, we_d, we_e = jnp.split(p['W_e'], 3, axis=0)
        if l == 0:
            wcm = params['W_ee'] @ p['W_msg']
            wce = params['W_ee'] @ we_e
            bcm = (params['b_ee'] @ p['W_msg']).reshape(1, H)
            bce = (params['b_ee'] @ we_e).reshape(1, H)
            em, ee = _edge_mm0(edge_attr, wcm, wce, bcm, bce)
        else:
            em, ee = _edge_mm(ee, g, params['layers'][l - 1]['b_e']
                              .reshape(1, H), p['W_msg'], we_e,
                              want_ee=(l < 2))
        hm = h @ p['W_msg']
        agg_flat = _sc_scatter(_split_cols(hm), em, src2, dst, zrows)
        agg = jnp.concatenate(
            [agg_flat[0:N], agg_flat[NPAD:NPAD + N]], axis=1)
        h = jax.nn.relu(h @ p['W_self'] + agg + p['b_h'])
        if l < 2:
            hs = h @ we_s
            hd = h @ we_d
            g = _sc_gather(_split_cols(hs), _split_cols(hd), src2, dst2, ident)
    return h @ params['W_pred'] + params['b_pred']
